# baseline (speedup 1.0000x reference)
import functools

import jax
import jax.numpy as jnp
from jax.experimental import pallas as pl
from jax.experimental.pallas import tpu as pltpu

N = 10000
BM = 400


def _mm_kernel(a_ref, b_ref, o_ref):
    o_ref[...] = jnp.dot(a_ref[...], b_ref[...],
                         preferred_element_type=jnp.float32)


def _gc1_kernel(adj_ref, s1_ref, b1_ref, w2_ref, s2_ref):
    h = jnp.dot(adj_ref[...], s1_ref[...],
                preferred_element_type=jnp.float32)
    h = jnp.maximum(h + b1_ref[...], 0.0)
    s2_ref[...] = jnp.dot(h, w2_ref[...],
                          preferred_element_type=jnp.float32)


def _gc2_kernel(adj_ref, s2_ref, b2_ref, o_ref):
    o_ref[...] = jnp.dot(adj_ref[...], s2_ref[...],
                         preferred_element_type=jnp.float32) + b2_ref[...]


@jax.jit
def kernel(x, adj, W1, b1, W2, b2):
    nfeat = x.shape[1]
    nhid = W1.shape[1]
    b1r = b1.reshape(1, nhid)
    b2r = b2.reshape(1, nfeat)

    s1 = pl.pallas_call(
        _mm_kernel,
        out_shape=jax.ShapeDtypeStruct((N, nhid), jnp.float32),
    )(x, W1)

    grid = (N // BM,)
    adj_spec = pl.BlockSpec((BM, N), lambda i: (i, 0))

    s2 = pl.pallas_call(
        _gc1_kernel,
        grid=grid,
        in_specs=[
            adj_spec,
            pl.BlockSpec((N, nhid), lambda i: (0, 0)),
            pl.BlockSpec((1, nhid), lambda i: (0, 0)),
            pl.BlockSpec((nhid, nfeat), lambda i: (0, 0)),
        ],
        out_specs=pl.BlockSpec((BM, nfeat), lambda i: (i, 0)),
        out_shape=jax.ShapeDtypeStruct((N, nfeat), jnp.float32),
        compiler_params=pltpu.CompilerParams(
            dimension_semantics=("arbitrary",),
        ),
    )(adj, s1, b1r, W2)

    out = pl.pallas_call(
        _gc2_kernel,
        grid=grid,
        in_specs=[
            adj_spec,
            pl.BlockSpec((N, nfeat), lambda i: (0, 0)),
            pl.BlockSpec((1, nfeat), lambda i: (0, 0)),
        ],
        out_specs=pl.BlockSpec((BM, nfeat), lambda i: (i, 0)),
        out_shape=jax.ShapeDtypeStruct((N, nfeat), jnp.float32),
        compiler_params=pltpu.CompilerParams(
            dimension_semantics=("arbitrary",),
        ),
    )(adj, s2, b2r)

    return out
